# Initial kernel scaffold; baseline (speedup 1.0000x reference)
import jax
import jax.numpy as jnp
from jax.experimental import pallas as pl


def kernel(x, edge_index, gnn_batch, W1, b1, W2, b2, W3, b3, W4, b4, w5, bi5, w6, bi6, Wc1, bc1, Wc2, bc2):
    raise NotImplementedError("write your pallas kernel here")


# speedup vs baseline: 1.0760x; 1.0760x over previous
import jax
import jax.numpy as jnp
from jax import lax
from jax.experimental import pallas as pl
from jax.experimental.pallas import tpu as pltpu
from jax.experimental.pallas import tpu_sc as plsc

EMBED = 128
SORTK = 30
NG = 11
N = 11000
E = 352000
FEAT = 385
FPAD = 512
PADN = 11264
DUMMY = 11000
NW = 32
NC = 2
CHUNK = 88
NCH = E // NW // CHUNK
RPW = PADN // 16

_f32 = jnp.float32


def _mask_src(src, dst):
    EP = 2752 * 128
    s2 = jnp.pad(src, (0, EP - E)).reshape(2752, 128)
    d2 = jnp.pad(dst, (0, EP - E), constant_values=1).reshape(2752, 128)

    def body(s_ref, d_ref, o_ref):
        o_ref[...] = jnp.where(s_ref[...] == d_ref[...], DUMMY, s_ref[...])

    out = pl.pallas_call(
        body,
        grid=(4,),
        in_specs=[pl.BlockSpec((688, 128), lambda i: (i, 0))] * 2,
        out_specs=pl.BlockSpec((688, 128), lambda i: (i, 0)),
        out_shape=jax.ShapeDtypeStruct((2752, 128), jnp.int32),
    )(s2, d2)
    return out.reshape(-1)[:E]


def _sc_edge_agg(table, srcm3, dst3, zeros):
    D = table.shape[1]
    mesh = plsc.VectorSubcoreMesh(core_axis_name="c", subcore_axis_name="s")

    def body(table_h, src_h, dst_h, zero_h, out_h, src_v, dst_v, rows_v, acc_sh, sem):
        c = lax.axis_index("c")
        s = lax.axis_index("s")
        gw = s * NC + c
        pltpu.sync_copy(src_h.at[gw], src_v)
        pltpu.sync_copy(dst_h.at[gw], dst_v)
        r0 = s * RPW
        pltpu.sync_copy(zero_h.at[pl.ds(r0, RPW)], acc_sh.at[pl.ds(r0, RPW)])
        plsc.subcore_barrier()

        def chunk(j, carry):
            pltpu.async_copy(table_h.at[src_v.at[j]], rows_v, sem).wait()
            pltpu.sync_copy(rows_v, acc_sh.at[dst_v.at[j]], add=True)
            return carry

        lax.fori_loop(0, NCH, chunk, 0)
        plsc.subcore_barrier()
        pltpu.sync_copy(acc_sh.at[pl.ds(r0, RPW)], out_h.at[c, pl.ds(r0, RPW)])

    fn = pl.kernel(
        body,
        out_type=jax.ShapeDtypeStruct((NC, PADN, D), _f32),
        mesh=mesh,
        compiler_params=pltpu.CompilerParams(use_tc_tiling_on_sc=False),
        scratch_types=[
            pltpu.VMEM((NCH, CHUNK), jnp.int32),
            pltpu.VMEM((NCH, CHUNK), jnp.int32),
            pltpu.VMEM((CHUNK, D), _f32),
            pltpu.VMEM_SHARED((PADN, D), _f32),
            pltpu.SemaphoreType.DMA,
        ],
    )
    return fn(table, srcm3, dst3, zeros)


def _sc_gather(xcat, sel):
    mesh = plsc.VectorSubcoreMesh(core_axis_name="c", subcore_axis_name="s")

    def body(tab_h, idx_h, out_h, idx_v, rows_v, sem):
        c = lax.axis_index("c")
        s = lax.axis_index("s")
        gw = s * NC + c
        pltpu.sync_copy(idx_h.at[gw], idx_v)
        pltpu.async_copy(tab_h.at[idx_v], rows_v, sem).wait()
        pltpu.sync_copy(rows_v, out_h.at[pl.ds(gw * 16, 16)])

    fn = pl.kernel(
        body,
        out_type=jax.ShapeDtypeStruct((512, FPAD), _f32),
        mesh=mesh,
        compiler_params=pltpu.CompilerParams(use_tc_tiling_on_sc=False),
        scratch_types=[
            pltpu.VMEM((16,), jnp.int32),
            pltpu.VMEM((16, FPAD), _f32),
            pltpu.SemaphoreType.DMA,
        ],
    )
    return fn(xcat, sel)


def _tc_topk(keys_r, batr):
    R = PADN // 128
    NEG = -3.0e38
    BIG = 1 << 30

    def body(k_ref, bat_ref, sel_ref):
        keys = k_ref[...]
        nid = (lax.broadcasted_iota(jnp.int32, (R, 128), 0) * 128
               + lax.broadcasted_iota(jnp.int32, (R, 128), 1))
        bat = bat_ref[...]
        lane = lax.broadcasted_iota(jnp.int32, (1, 128), 1)

        def per_graph(g, carry):
            kg = jnp.where(bat == g, keys, NEG)
            row0 = jnp.full((1, 128), DUMMY, jnp.int32)

            def step(t, st):
                kg, row = st
                m = jnp.max(kg)
                idx = jnp.min(jnp.where(kg == m, nid, BIG))
                idx = jnp.where(m > NEG, idx, DUMMY)
                row = jnp.where(lane == t, idx, row)
                kg = jnp.where(nid == idx, NEG, kg)
                return kg, row

            _, row = lax.fori_loop(0, SORTK, step, (kg, row0))
            sel_ref[pl.ds(g, 1), :] = row
            return carry

        lax.fori_loop(0, NG, per_graph, 0)

    return pl.pallas_call(
        body,
        out_shape=jax.ShapeDtypeStruct((NG, 128), jnp.int32),
    )(keys_r, batr)


def _dot_hp(a, b):
    a0 = a.astype(jnp.bfloat16)
    a1 = (a - a0.astype(_f32)).astype(jnp.bfloat16)
    b0 = b.astype(jnp.bfloat16)
    b1 = (b - b0.astype(_f32)).astype(jnp.bfloat16)

    def d(u, v):
        return jnp.dot(u, v, preferred_element_type=_f32)

    return ((d(a0, b0) + d(a1, b1)) + (d(a0, b1) + d(a1, b0)))


def _tc_h1pool(selrows, w5m, b5):
    def body(a_ref, w_ref, b_ref, o_ref):
        h1 = jnp.maximum(_dot_hp(a_ref[...], w_ref[...]) + b_ref[...], 0.0)
        o_ref[...] = jnp.max(h1[0:330].reshape(165, 2, 64), axis=1)

    return pl.pallas_call(
        body,
        out_shape=jax.ShapeDtypeStruct((165, 64), _f32),
    )(selrows, w5m, b5)


def _tc_matmul_relu(A, W, b):
    def body(a_ref, w_ref, b_ref, o_ref):
        o_ref[...] = jnp.maximum(
            _dot_hp(a_ref[...], w_ref[...]) + b_ref[...], 0.0)

    return pl.pallas_call(
        body,
        out_shape=jax.ShapeDtypeStruct((A.shape[0], W.shape[1]), _f32),
    )(A, W, b)


def _tc_final(flat, Wc1r, bc1, Wc2, bc2):
    def body(f_ref, w1_ref, b1_ref, w2_ref, b2_ref, o_ref):
        hid = jnp.maximum(
            _dot_hp(f_ref[...], w1_ref[...]) + b1_ref[...], 0.0)
        o_ref[...] = _dot_hp(hid, w2_ref[...]) + b2_ref[...]

    return pl.pallas_call(
        body,
        out_shape=jax.ShapeDtypeStruct((1, 10), _f32),
    )(flat, Wc1r, bc1, Wc2, bc2)


def kernel(x, edge_index, gnn_batch, W1, b1, W2, b2, W3, b3, W4, b4,
           w5, bi5, w6, bi6, Wc1, bc1, Wc2, bc2):
    src = edge_index[0]
    dst = edge_index[1]

    srcm = _mask_src(src, dst)
    srcm3 = srcm.reshape(NW, NCH, CHUNK)
    dst3 = dst.reshape(NW, NCH, CHUNK)
    ones8 = jnp.pad(jnp.ones((N, 8), _f32), ((0, PADN - N), (0, 0)))
    zeros8 = jnp.zeros((PADN, 8), _f32)
    dacc = _sc_edge_agg(ones8, srcm3, dst3, zeros8)
    deg = dacc[0, :N, 0] + dacc[1, :N, 0] + 1.0
    dinv = lax.rsqrt(deg)
    sinv = (1.0 / deg)

    w_e = jnp.where(src == dst, 0.0, 1.0).astype(_f32)
    norm = dinv[src] * dinv[dst] * w_e

    def gcn(xl, W, b):
        xw = xl @ W
        out = jnp.zeros_like(xw).at[dst].add(norm[:, None] * xw[src])
        out = out + xw * sinv[:, None]
        return jnp.tanh(out + b)

    x1 = gcn(x, W1, b1)
    x2 = gcn(x1, W2, b2)
    x3 = gcn(x2, W3, b3)
    x4 = gcn(x3, W4, b4)

    R = PADN // 128
    keys_r = jnp.pad(x4[:, 0], (0, PADN - N)).reshape(R, 128)
    batr = jnp.pad(gnn_batch, (0, PADN - N),
                   constant_values=127).reshape(R, 128)
    selout = _tc_topk(keys_r, batr)

    xcat = jnp.pad(jnp.concatenate([x1, x2, x3, x4], axis=1),
                   ((0, PADN - N), (0, FPAD - FEAT)))
    sel = jnp.concatenate(
        [selout[:, :SORTK].reshape(-1),
         jnp.full((512 - NG * SORTK,), DUMMY, jnp.int32)]).reshape(32, 16)
    selrows = _sc_gather(xcat, sel)

    w5m = jnp.pad(w5[:, 0, :].T, ((0, FPAD - FEAT), (0, 0)))
    hp = _tc_h1pool(selrows, w5m, bi5.reshape(1, 64))
    hpg = hp.reshape(NG, 15, 64)
    X2 = jnp.concatenate(
        [hpg[:, k:k + 11, :] for k in range(5)], axis=2).reshape(121, 320)
    w6m = jnp.transpose(w6, (2, 1, 0)).reshape(320, 128)
    h2 = _tc_matmul_relu(X2, w6m, bi6.reshape(1, 128))
    flat = h2.reshape(1, NG * 128 * 11)
    Wc1r = jnp.transpose(
        Wc1.reshape(NG, 128, 11, 128), (0, 2, 1, 3)).reshape(NG * 128 * 11, 128)
    return _tc_final(flat, Wc1r, bc1.reshape(1, 128), Wc2, bc2.reshape(1, 10))


# speedup vs baseline: 1.4078x; 1.3084x over previous
import jax
import jax.numpy as jnp
from jax import lax
from jax.experimental import pallas as pl
from jax.experimental.pallas import tpu as pltpu
from jax.experimental.pallas import tpu_sc as plsc

EMBED = 128
SORTK = 30
NG = 11
N = 11000
E = 352000
FEAT = 385
FPAD = 512
PADN = 11264
DUMMY = 11000
NW = 32
NC = 2
CHUNK = 88
NCH = E // NW // CHUNK
RPW = PADN // 16

_f32 = jnp.float32


def _mask_src(src, dst):
    EP = 2752 * 128
    s2 = jnp.pad(src, (0, EP - E)).reshape(2752, 128)
    d2 = jnp.pad(dst, (0, EP - E), constant_values=1).reshape(2752, 128)

    def body(s_ref, d_ref, o_ref):
        o_ref[...] = jnp.where(s_ref[...] == d_ref[...], DUMMY, s_ref[...])

    out = pl.pallas_call(
        body,
        grid=(4,),
        in_specs=[pl.BlockSpec((688, 128), lambda i: (i, 0))] * 2,
        out_specs=pl.BlockSpec((688, 128), lambda i: (i, 0)),
        out_shape=jax.ShapeDtypeStruct((2752, 128), jnp.int32),
    )(s2, d2)
    return out.reshape(-1)[:E]


def _sc_edge_agg(table, srcm3, dst3, zeros):
    D = table.shape[1]
    mesh = plsc.VectorSubcoreMesh(core_axis_name="c", subcore_axis_name="s")

    def body(table_h, src_h, dst_h, zero_h, out_h, src_v, dst_v, rows_v, acc_sh, sem):
        c = lax.axis_index("c")
        s = lax.axis_index("s")
        gw = s * NC + c
        pltpu.sync_copy(src_h.at[gw], src_v)
        pltpu.sync_copy(dst_h.at[gw], dst_v)
        r0 = s * RPW
        pltpu.sync_copy(zero_h.at[pl.ds(r0, RPW)], acc_sh.at[pl.ds(r0, RPW)])
        plsc.subcore_barrier()

        def chunk(j, carry):
            pltpu.async_copy(table_h.at[src_v.at[j]], rows_v, sem).wait()
            pltpu.sync_copy(rows_v, acc_sh.at[dst_v.at[j]], add=True)
            return carry

        lax.fori_loop(0, NCH, chunk, 0)
        plsc.subcore_barrier()
        pltpu.sync_copy(acc_sh.at[pl.ds(r0, RPW)], out_h.at[c, pl.ds(r0, RPW)])

    fn = pl.kernel(
        body,
        out_type=jax.ShapeDtypeStruct((NC, PADN, D), _f32),
        mesh=mesh,
        compiler_params=pltpu.CompilerParams(use_tc_tiling_on_sc=False),
        scratch_types=[
            pltpu.VMEM((NCH, CHUNK), jnp.int32),
            pltpu.VMEM((NCH, CHUNK), jnp.int32),
            pltpu.VMEM((CHUNK, D), _f32),
            pltpu.VMEM_SHARED((PADN, D), _f32),
            pltpu.SemaphoreType.DMA,
        ],
    )
    return fn(table, srcm3, dst3, zeros)


def _sc_edge_gather(table, src3):
    D = table.shape[1]
    EPW = E // NW
    mesh = plsc.VectorSubcoreMesh(core_axis_name="c", subcore_axis_name="s")

    def body(tab_h, src_h, out_h, src_v, rows_a, rows_b, sga, sgb, swa, swb):
        c = lax.axis_index("c")
        s = lax.axis_index("s")
        gw = s * NC + c
        pltpu.sync_copy(src_h.at[gw], src_v)
        base = gw * EPW
        pltpu.async_copy(tab_h.at[src_v.at[0]], rows_a, sga).wait()
        pltpu.sync_copy(rows_a, out_h.at[pl.ds(base, CHUNK)])
        pltpu.async_copy(tab_h.at[src_v.at[1]], rows_a, sga)
        pltpu.async_copy(tab_h.at[src_v.at[2]], rows_b, sgb)

        def pair(i, carry):
            j = 1 + 2 * i
            pltpu.make_async_copy(tab_h.at[src_v.at[j]], rows_a, sga).wait()
            wa = pltpu.async_copy(
                rows_a, out_h.at[pl.ds(base + j * CHUNK, CHUNK)], swa)
            pltpu.make_async_copy(tab_h.at[src_v.at[j + 1]], rows_b, sgb).wait()
            wb = pltpu.async_copy(
                rows_b, out_h.at[pl.ds(base + (j + 1) * CHUNK, CHUNK)], swb)
            wa.wait()

            @pl.when(j + 2 < NCH)
            def _():
                pltpu.async_copy(tab_h.at[src_v.at[j + 2]], rows_a, sga)

            wb.wait()

            @pl.when(j + 3 < NCH)
            def _():
                pltpu.async_copy(tab_h.at[src_v.at[j + 3]], rows_b, sgb)

            return carry

        lax.fori_loop(0, (NCH - 1) // 2, pair, 0)

    fn = pl.kernel(
        body,
        out_type=jax.ShapeDtypeStruct((E, D), _f32),
        mesh=mesh,
        compiler_params=pltpu.CompilerParams(use_tc_tiling_on_sc=False),
        scratch_types=[
            pltpu.VMEM((NCH, CHUNK), jnp.int32),
            pltpu.VMEM((CHUNK, D), _f32),
            pltpu.VMEM((CHUNK, D), _f32),
            pltpu.SemaphoreType.DMA,
            pltpu.SemaphoreType.DMA,
            pltpu.SemaphoreType.DMA,
            pltpu.SemaphoreType.DMA,
        ],
    )
    return fn(table, src3)


def _sc_gather(xcat, sel):
    mesh = plsc.VectorSubcoreMesh(core_axis_name="c", subcore_axis_name="s")

    def body(tab_h, idx_h, out_h, idx_v, rows_v, sem):
        c = lax.axis_index("c")
        s = lax.axis_index("s")
        gw = s * NC + c
        pltpu.sync_copy(idx_h.at[gw], idx_v)
        pltpu.async_copy(tab_h.at[idx_v], rows_v, sem).wait()
        pltpu.sync_copy(rows_v, out_h.at[pl.ds(gw * 16, 16)])

    fn = pl.kernel(
        body,
        out_type=jax.ShapeDtypeStruct((512, FPAD), _f32),
        mesh=mesh,
        compiler_params=pltpu.CompilerParams(use_tc_tiling_on_sc=False),
        scratch_types=[
            pltpu.VMEM((16,), jnp.int32),
            pltpu.VMEM((16, FPAD), _f32),
            pltpu.SemaphoreType.DMA,
        ],
    )
    return fn(xcat, sel)


def _tc_topk(keys_r, batr):
    R = PADN // 128
    NEG = -3.0e38
    BIG = 1 << 30

    def body(k_ref, bat_ref, sel_ref):
        keys = k_ref[...]
        nid = (lax.broadcasted_iota(jnp.int32, (R, 128), 0) * 128
               + lax.broadcasted_iota(jnp.int32, (R, 128), 1))
        bat = bat_ref[...]
        lane = lax.broadcasted_iota(jnp.int32, (1, 128), 1)

        def per_graph(g, carry):
            kg = jnp.where(bat == g, keys, NEG)
            row0 = jnp.full((1, 128), DUMMY, jnp.int32)

            def step(t, st):
                kg, row = st
                m = jnp.max(kg)
                idx = jnp.min(jnp.where(kg == m, nid, BIG))
                idx = jnp.where(m > NEG, idx, DUMMY)
                row = jnp.where(lane == t, idx, row)
                kg = jnp.where(nid == idx, NEG, kg)
                return kg, row

            _, row = lax.fori_loop(0, SORTK, step, (kg, row0))
            sel_ref[pl.ds(g, 1), :] = row
            return carry

        lax.fori_loop(0, NG, per_graph, 0)

    return pl.pallas_call(
        body,
        out_shape=jax.ShapeDtypeStruct((NG, 128), jnp.int32),
    )(keys_r, batr)


def _dot_hp(a, b):
    a0 = a.astype(jnp.bfloat16)
    a1 = (a - a0.astype(_f32)).astype(jnp.bfloat16)
    b0 = b.astype(jnp.bfloat16)
    b1 = (b - b0.astype(_f32)).astype(jnp.bfloat16)

    def d(u, v):
        return jnp.dot(u, v, preferred_element_type=_f32)

    return ((d(a0, b0) + d(a1, b1)) + (d(a0, b1) + d(a1, b0)))


def _tc_h1pool(selrows, w5m, b5):
    def body(a_ref, w_ref, b_ref, o_ref):
        h1 = jnp.maximum(_dot_hp(a_ref[...], w_ref[...]) + b_ref[...], 0.0)
        o_ref[...] = jnp.max(h1[0:330].reshape(165, 2, 64), axis=1)

    return pl.pallas_call(
        body,
        out_shape=jax.ShapeDtypeStruct((165, 64), _f32),
    )(selrows, w5m, b5)


def _tc_matmul_relu(A, W, b):
    def body(a_ref, w_ref, b_ref, o_ref):
        o_ref[...] = jnp.maximum(
            _dot_hp(a_ref[...], w_ref[...]) + b_ref[...], 0.0)

    return pl.pallas_call(
        body,
        out_shape=jax.ShapeDtypeStruct((A.shape[0], W.shape[1]), _f32),
    )(A, W, b)


def _tc_final(flat, Wc1r, bc1, Wc2, bc2):
    def body(f_ref, w1_ref, b1_ref, w2_ref, b2_ref, o_ref):
        hid = jnp.maximum(
            _dot_hp(f_ref[...], w1_ref[...]) + b1_ref[...], 0.0)
        o_ref[...] = _dot_hp(hid, w2_ref[...]) + b2_ref[...]

    return pl.pallas_call(
        body,
        out_shape=jax.ShapeDtypeStruct((1, 10), _f32),
    )(flat, Wc1r, bc1, Wc2, bc2)


def kernel(x, edge_index, gnn_batch, W1, b1, W2, b2, W3, b3, W4, b4,
           w5, bi5, w6, bi6, Wc1, bc1, Wc2, bc2):
    src = edge_index[0]
    dst = edge_index[1]

    srcm = _mask_src(src, dst)
    srcm3 = srcm.reshape(NW, NCH, CHUNK)
    dst3 = dst.reshape(NW, NCH, CHUNK)
    ones8 = jnp.pad(jnp.ones((N, 8), _f32), ((0, PADN - N), (0, 0)))
    zeros8 = jnp.zeros((PADN, 8), _f32)
    dacc = _sc_edge_agg(ones8, srcm3, dst3, zeros8)
    deg = dacc[0, :N, 0] + dacc[1, :N, 0] + 1.0
    dinv = lax.rsqrt(deg)
    sinv = (1.0 / deg)

    w_e = jnp.where(src == dst, 0.0, 1.0).astype(_f32)
    norm = dinv[src] * dinv[dst] * w_e
    src3 = src.reshape(NW, NCH, CHUNK)

    def gcn(xl, W, b):
        xw = xl @ W
        if W.shape[1] == EMBED:
            g = _sc_edge_gather(xw, src3)
        else:
            g = _sc_edge_gather(jnp.tile(xw, (1, 8)), src3)[:, 0:1]
        out = jnp.zeros_like(xw).at[dst].add(norm[:, None] * g)
        out = out + xw * sinv[:, None]
        return jnp.tanh(out + b)

    x1 = gcn(x, W1, b1)
    x2 = gcn(x1, W2, b2)
    x3 = gcn(x2, W3, b3)
    x4 = gcn(x3, W4, b4)

    R = PADN // 128
    keys_r = jnp.pad(x4[:, 0], (0, PADN - N)).reshape(R, 128)
    batr = jnp.pad(gnn_batch, (0, PADN - N),
                   constant_values=127).reshape(R, 128)
    selout = _tc_topk(keys_r, batr)

    xcat = jnp.pad(jnp.concatenate([x1, x2, x3, x4], axis=1),
                   ((0, PADN - N), (0, FPAD - FEAT)))
    sel = jnp.concatenate(
        [selout[:, :SORTK].reshape(-1),
         jnp.full((512 - NG * SORTK,), DUMMY, jnp.int32)]).reshape(32, 16)
    selrows = _sc_gather(xcat, sel)

    w5m = jnp.pad(w5[:, 0, :].T, ((0, FPAD - FEAT), (0, 0)))
    hp = _tc_h1pool(selrows, w5m, bi5.reshape(1, 64))
    hpg = hp.reshape(NG, 15, 64)
    X2 = jnp.concatenate(
        [hpg[:, k:k + 11, :] for k in range(5)], axis=2).reshape(121, 320)
    w6m = jnp.transpose(w6, (2, 1, 0)).reshape(320, 128)
    h2 = _tc_matmul_relu(X2, w6m, bi6.reshape(1, 128))
    flat = h2.reshape(1, NG * 128 * 11)
    Wc1r = jnp.transpose(
        Wc1.reshape(NG, 128, 11, 128), (0, 2, 1, 3)).reshape(NG * 128 * 11, 128)
    return _tc_final(flat, Wc1r, bc1.reshape(1, 128), Wc2, bc2.reshape(1, 10))


# speedup vs baseline: 2.6463x; 1.8797x over previous
import jax
import jax.numpy as jnp
from jax import lax
from jax.experimental import pallas as pl
from jax.experimental.pallas import tpu as pltpu
from jax.experimental.pallas import tpu_sc as plsc

EMBED = 128
SORTK = 30
NG = 11
N = 11000
E = 352000
FEAT = 385
FPAD = 512
PADN = 11264
DUMMY = 11000
NW = 32
NC = 2
CHUNK = 88
NCH = E // NW // CHUNK
RPW = PADN // 16

_f32 = jnp.float32


def _mask_src(src, dst):
    EP = 2752 * 128
    s2 = jnp.pad(src, (0, EP - E)).reshape(2752, 128)
    d2 = jnp.pad(dst, (0, EP - E), constant_values=1).reshape(2752, 128)

    def body(s_ref, d_ref, o_ref):
        o_ref[...] = jnp.where(s_ref[...] == d_ref[...], DUMMY, s_ref[...])

    out = pl.pallas_call(
        body,
        grid=(4,),
        in_specs=[pl.BlockSpec((688, 128), lambda i: (i, 0))] * 2,
        out_specs=pl.BlockSpec((688, 128), lambda i: (i, 0)),
        out_shape=jax.ShapeDtypeStruct((2752, 128), jnp.int32),
    )(s2, d2)
    return out.reshape(-1)[:E]


def _sc_edge_agg(table, srcm3, dst3, zeros):
    D = table.shape[1]
    mesh = plsc.VectorSubcoreMesh(core_axis_name="c", subcore_axis_name="s")

    def body(table_h, src_h, dst_h, zero_h, out_h, src_v, dst_v, rows_v, acc_sh, sem):
        c = lax.axis_index("c")
        s = lax.axis_index("s")
        gw = s * NC + c
        pltpu.sync_copy(src_h.at[gw], src_v)
        pltpu.sync_copy(dst_h.at[gw], dst_v)
        r0 = s * RPW
        pltpu.sync_copy(zero_h.at[pl.ds(r0, RPW)], acc_sh.at[pl.ds(r0, RPW)])
        plsc.subcore_barrier()

        def chunk(j, carry):
            pltpu.async_copy(table_h.at[src_v.at[j]], rows_v, sem).wait()
            pltpu.sync_copy(rows_v, acc_sh.at[dst_v.at[j]], add=True)
            return carry

        lax.fori_loop(0, NCH, chunk, 0)
        plsc.subcore_barrier()
        pltpu.sync_copy(acc_sh.at[pl.ds(r0, RPW)], out_h.at[c, pl.ds(r0, RPW)])

    fn = pl.kernel(
        body,
        out_type=jax.ShapeDtypeStruct((NC, PADN, D), _f32),
        mesh=mesh,
        compiler_params=pltpu.CompilerParams(use_tc_tiling_on_sc=False),
        scratch_types=[
            pltpu.VMEM((NCH, CHUNK), jnp.int32),
            pltpu.VMEM((NCH, CHUNK), jnp.int32),
            pltpu.VMEM((CHUNK, D), _f32),
            pltpu.VMEM_SHARED((PADN, D), _f32),
            pltpu.SemaphoreType.DMA,
        ],
    )
    return fn(table, srcm3, dst3, zeros)


def _sc_edge_gather(table, src3):
    D = table.shape[1]
    EPW = E // NW
    mesh = plsc.VectorSubcoreMesh(core_axis_name="c", subcore_axis_name="s")

    def body(tab_h, src_h, out_h, src_v, rows_a, rows_b, sga, sgb, swa, swb):
        c = lax.axis_index("c")
        s = lax.axis_index("s")
        gw = s * NC + c
        pltpu.sync_copy(src_h.at[gw], src_v)
        base = gw * EPW
        pltpu.async_copy(tab_h.at[src_v.at[0]], rows_a, sga).wait()
        pltpu.sync_copy(rows_a, out_h.at[pl.ds(base, CHUNK)])
        pltpu.async_copy(tab_h.at[src_v.at[1]], rows_a, sga)
        pltpu.async_copy(tab_h.at[src_v.at[2]], rows_b, sgb)

        def pair(i, carry):
            j = 1 + 2 * i
            pltpu.make_async_copy(tab_h.at[src_v.at[j]], rows_a, sga).wait()
            wa = pltpu.async_copy(
                rows_a, out_h.at[pl.ds(base + j * CHUNK, CHUNK)], swa)
            pltpu.make_async_copy(tab_h.at[src_v.at[j + 1]], rows_b, sgb).wait()
            wb = pltpu.async_copy(
                rows_b, out_h.at[pl.ds(base + (j + 1) * CHUNK, CHUNK)], swb)
            wa.wait()

            @pl.when(j + 2 < NCH)
            def _():
                pltpu.async_copy(tab_h.at[src_v.at[j + 2]], rows_a, sga)

            wb.wait()

            @pl.when(j + 3 < NCH)
            def _():
                pltpu.async_copy(tab_h.at[src_v.at[j + 3]], rows_b, sgb)

            return carry

        lax.fori_loop(0, (NCH - 1) // 2, pair, 0)

    fn = pl.kernel(
        body,
        out_type=jax.ShapeDtypeStruct((E, D), _f32),
        mesh=mesh,
        compiler_params=pltpu.CompilerParams(use_tc_tiling_on_sc=False),
        scratch_types=[
            pltpu.VMEM((NCH, CHUNK), jnp.int32),
            pltpu.VMEM((CHUNK, D), _f32),
            pltpu.VMEM((CHUNK, D), _f32),
            pltpu.SemaphoreType.DMA,
            pltpu.SemaphoreType.DMA,
            pltpu.SemaphoreType.DMA,
            pltpu.SemaphoreType.DMA,
        ],
    )
    return fn(table, src3)


def _sc_gather(xcat, sel):
    mesh = plsc.VectorSubcoreMesh(core_axis_name="c", subcore_axis_name="s")

    def body(tab_h, idx_h, out_h, idx_v, rows_v, sem):
        c = lax.axis_index("c")
        s = lax.axis_index("s")
        gw = s * NC + c
        pltpu.sync_copy(idx_h.at[gw], idx_v)
        pltpu.async_copy(tab_h.at[idx_v], rows_v, sem).wait()
        pltpu.sync_copy(rows_v, out_h.at[pl.ds(gw * 16, 16)])

    fn = pl.kernel(
        body,
        out_type=jax.ShapeDtypeStruct((512, FPAD), _f32),
        mesh=mesh,
        compiler_params=pltpu.CompilerParams(use_tc_tiling_on_sc=False),
        scratch_types=[
            pltpu.VMEM((16,), jnp.int32),
            pltpu.VMEM((16, FPAD), _f32),
            pltpu.SemaphoreType.DMA,
        ],
    )
    return fn(xcat, sel)


def _tc_topk(keys_r, batr):
    R = PADN // 128
    NEG = -3.0e38
    BIG = 1 << 30

    def body(k_ref, bat_ref, sel_ref):
        keys = k_ref[...]
        nid = (lax.broadcasted_iota(jnp.int32, (R, 128), 0) * 128
               + lax.broadcasted_iota(jnp.int32, (R, 128), 1))
        bat = bat_ref[...]
        lane = lax.broadcasted_iota(jnp.int32, (1, 128), 1)

        def per_graph(g, carry):
            kg = jnp.where(bat == g, keys, NEG)
            row0 = jnp.full((1, 128), DUMMY, jnp.int32)

            def step(t, st):
                kg, row = st
                m = jnp.max(kg)
                idx = jnp.min(jnp.where(kg == m, nid, BIG))
                idx = jnp.where(m > NEG, idx, DUMMY)
                row = jnp.where(lane == t, idx, row)
                kg = jnp.where(nid == idx, NEG, kg)
                return kg, row

            _, row = lax.fori_loop(0, SORTK, step, (kg, row0))
            sel_ref[pl.ds(g, 1), :] = row
            return carry

        lax.fori_loop(0, NG, per_graph, 0)

    return pl.pallas_call(
        body,
        out_shape=jax.ShapeDtypeStruct((NG, 128), jnp.int32),
    )(keys_r, batr)


def _dot_hp(a, b):
    a0 = a.astype(jnp.bfloat16)
    a1 = (a - a0.astype(_f32)).astype(jnp.bfloat16)
    b0 = b.astype(jnp.bfloat16)
    b1 = (b - b0.astype(_f32)).astype(jnp.bfloat16)

    def d(u, v):
        return jnp.dot(u, v, preferred_element_type=_f32)

    return ((d(a0, b0) + d(a1, b1)) + (d(a0, b1) + d(a1, b0)))


def _tc_h1pool(selrows, w5m, b5):
    def body(a_ref, w_ref, b_ref, o_ref):
        h1 = jnp.maximum(_dot_hp(a_ref[...], w_ref[...]) + b_ref[...], 0.0)
        o_ref[...] = jnp.max(h1[0:330].reshape(165, 2, 64), axis=1)

    return pl.pallas_call(
        body,
        out_shape=jax.ShapeDtypeStruct((165, 64), _f32),
    )(selrows, w5m, b5)


def _tc_matmul_relu(A, W, b):
    def body(a_ref, w_ref, b_ref, o_ref):
        o_ref[...] = jnp.maximum(
            _dot_hp(a_ref[...], w_ref[...]) + b_ref[...], 0.0)

    return pl.pallas_call(
        body,
        out_shape=jax.ShapeDtypeStruct((A.shape[0], W.shape[1]), _f32),
    )(A, W, b)


def _tc_final(flat, Wc1r, bc1, Wc2, bc2):
    def body(f_ref, w1_ref, b1_ref, w2_ref, b2_ref, o_ref):
        hid = jnp.maximum(
            _dot_hp(f_ref[...], w1_ref[...]) + b1_ref[...], 0.0)
        o_ref[...] = _dot_hp(hid, w2_ref[...]) + b2_ref[...]

    return pl.pallas_call(
        body,
        out_shape=jax.ShapeDtypeStruct((1, 10), _f32),
    )(flat, Wc1r, bc1, Wc2, bc2)


def kernel(x, edge_index, gnn_batch, W1, b1, W2, b2, W3, b3, W4, b4,
           w5, bi5, w6, bi6, Wc1, bc1, Wc2, bc2):
    src = edge_index[0]
    dst = edge_index[1]

    srcm = _mask_src(src, dst)
    srcm3 = srcm.reshape(NW, NCH, CHUNK)
    dst3 = dst.reshape(NW, NCH, CHUNK)
    ones8 = jnp.pad(jnp.ones((N, 8), _f32), ((0, PADN - N), (0, 0)))
    zeros8 = jnp.zeros((PADN, 8), _f32)
    dacc = _sc_edge_agg(ones8, srcm3, dst3, zeros8)
    deg = dacc[0, :N, 0] + dacc[1, :N, 0] + 1.0
    dinv = lax.rsqrt(deg)
    sinv = (1.0 / deg)

    w_e = jnp.where(src == dst, 0.0, 1.0).astype(_f32)
    src3 = src.reshape(NW, NCH, CHUNK)
    dinv8 = jnp.tile(dinv[:, None], (1, 8))
    dinv_src = _sc_edge_gather(dinv8, src3)[:, 0]
    dinv_dst = _sc_edge_gather(dinv8, dst3)[:, 0]
    norm = dinv_src * dinv_dst * w_e

    def gcn(xl, W, b):
        xw = xl @ W
        if W.shape[1] == EMBED:
            g = _sc_edge_gather(xw, src3)
        else:
            g = _sc_edge_gather(jnp.tile(xw, (1, 8)), src3)[:, 0:1]
        out = jnp.zeros_like(xw).at[dst].add(norm[:, None] * g)
        out = out + xw * sinv[:, None]
        return jnp.tanh(out + b)

    x1 = gcn(x, W1, b1)
    x2 = gcn(x1, W2, b2)
    x3 = gcn(x2, W3, b3)
    x4 = gcn(x3, W4, b4)

    R = PADN // 128
    keys_r = jnp.pad(x4[:, 0], (0, PADN - N)).reshape(R, 128)
    batr = jnp.pad(gnn_batch, (0, PADN - N),
                   constant_values=127).reshape(R, 128)
    selout = _tc_topk(keys_r, batr)

    xcat = jnp.pad(jnp.concatenate([x1, x2, x3, x4], axis=1),
                   ((0, PADN - N), (0, FPAD - FEAT)))
    sel = jnp.concatenate(
        [selout[:, :SORTK].reshape(-1),
         jnp.full((512 - NG * SORTK,), DUMMY, jnp.int32)]).reshape(32, 16)
    selrows = _sc_gather(xcat, sel)

    w5m = jnp.pad(w5[:, 0, :].T, ((0, FPAD - FEAT), (0, 0)))
    hp = _tc_h1pool(selrows, w5m, bi5.reshape(1, 64))
    hpg = hp.reshape(NG, 15, 64)
    X2 = jnp.concatenate(
        [hpg[:, k:k + 11, :] for k in range(5)], axis=2).reshape(121, 320)
    w6m = jnp.transpose(w6, (2, 1, 0)).reshape(320, 128)
    h2 = _tc_matmul_relu(X2, w6m, bi6.reshape(1, 128))
    flat = h2.reshape(1, NG * 128 * 11)
    Wc1r = jnp.transpose(
        Wc1.reshape(NG, 128, 11, 128), (0, 2, 1, 3)).reshape(NG * 128 * 11, 128)
    return _tc_final(flat, Wc1r, bc1.reshape(1, 128), Wc2, bc2.reshape(1, 10))
